# baseline (speedup 1.0000x reference)
import functools

import jax
import jax.numpy as jnp
from jax import lax
from jax.experimental import pallas as pl
from jax.experimental.pallas import tpu as pltpu
from jax.experimental.pallas import tpu_sc as plsc

_NEG = -1e9
_ROW_BLK = 512


def _argmax_body(q_ref, k_ref, fcol_ref, fq_ref, out_ref):
    q = q_ref[...]
    k = k_ref[...]
    norms = jnp.sqrt(jnp.sum(k * k, axis=1, keepdims=True)) + 1e-8
    kn = k / norms
    sim = jax.lax.dot_general(
        q, kn, (((1,), (1,)), ((), ())),
        preferred_element_type=jnp.float32)
    fcol = fcol_ref[...]
    sim = jnp.where(fcol >= 1, _NEG, sim)
    m = jnp.max(sim, axis=1, keepdims=True)
    ids = lax.broadcasted_iota(jnp.int32, sim.shape, 1)
    cand = jnp.where(sim == m, ids, jnp.int32(2**30))
    idx = jnp.min(cand, axis=1, keepdims=True)
    fq = fq_ref[0]
    n_total = k.shape[0]
    out_ref[0] = jnp.where(fq >= 1, idx, jnp.int32(n_total))


def _compute_idx(latter0, flag):
    n, c2 = latter0.shape
    nblk = n // _ROW_BLK
    fcol = flag.reshape(1, n)
    fq = flag.reshape(nblk, _ROW_BLK, 1)
    grid_spec = pl.GridSpec(
        grid=(nblk,),
        in_specs=[
            pl.BlockSpec((_ROW_BLK, c2), lambda i: (i, 0)),
            pl.BlockSpec((n, c2), lambda i: (0, 0)),
            pl.BlockSpec((1, n), lambda i: (0, 0)),
            pl.BlockSpec((1, _ROW_BLK, 1), lambda i: (i, 0, 0)),
        ],
        out_specs=pl.BlockSpec((1, _ROW_BLK, 1), lambda i: (i, 0, 0)),
    )
    out = pl.pallas_call(
        _argmax_body,
        grid_spec=grid_spec,
        out_shape=jax.ShapeDtypeStruct((nblk, _ROW_BLK, 1), jnp.int32),
    )(latter0, latter0, fcol, fq)
    return out.reshape(n)


def _sc_gather(table, idx):
    n = idx.shape[0]
    c2 = table.shape[1]
    info = plsc.get_sparse_core_info()
    nc, ns = info.num_cores, info.num_subcores
    nw = nc * ns
    b_per_w = n // nw
    mesh = plsc.VectorSubcoreMesh(core_axis_name="c", subcore_axis_name="s")

    @functools.partial(
        pl.kernel, mesh=mesh,
        out_type=jax.ShapeDtypeStruct((n, c2), jnp.float32),
        scratch_types=[
            pltpu.VMEM((b_per_w,), jnp.int32),
            pltpu.VMEM((b_per_w, c2), jnp.float32),
            pltpu.SemaphoreType.DMA,
        ],
    )
    def gather_k(table_hbm, idx_hbm, out_hbm, idx_v, rows_v, sem):
        wid = lax.axis_index("s") * nc + lax.axis_index("c")
        base = wid * b_per_w
        pltpu.sync_copy(idx_hbm.at[pl.ds(base, b_per_w)], idx_v)
        pltpu.async_copy(table_hbm.at[idx_v], rows_v, sem).wait()
        pltpu.sync_copy(rows_v, out_hbm.at[pl.ds(base, b_per_w)])

    return gather_k(table, idx)


def kernel(input, mask):
    b, c, h, w = input.shape
    c2 = c // 2
    n = h * w
    former = input[:, :c2]
    latter = input[:, c2:]
    latter0 = latter[0].reshape(c2, n).T
    former0 = former[0].reshape(c2, n).T
    flag = mask.reshape(n).astype(jnp.int32)

    idx = _compute_idx(latter0, flag)

    v_pad = ((n + 1 + 7) // 8) * 8
    d_pad = 128
    table = jnp.zeros((v_pad, d_pad), jnp.float32).at[:n, :c2].set(former0)
    shift = _sc_gather(table, idx)[:, :c2]

    shift_map = jnp.broadcast_to(shift.T.reshape(1, c2, h, w), (b, c2, h, w))
    return jnp.concatenate([former, latter, shift_map], axis=1)
